# Initial kernel scaffold; baseline (speedup 1.0000x reference)
import jax
import jax.numpy as jnp
from jax.experimental import pallas as pl


def kernel(a_row, a_col, a_val, x_row, x_col, x_val, emb, W1, W2, W3, ln_g, ln_b, Wm1, bm1, bn1_g, bn1_b, bn1_m, bn1_v, Wm2, bm2, bn2_g, bn2_b, bn2_m, bn2_v, Wc, bc):
    raise NotImplementedError("write your pallas kernel here")


# speedup vs baseline: 8.2252x; 8.2252x over previous
import jax
import jax.numpy as jnp
from jax import lax
from jax.experimental import pallas as pl
from jax.experimental.pallas import tpu as pltpu
from jax.experimental.pallas import tpu_sc as plsc

_HID = 128
_CH = 128
_KC = 8
_WIN = _CH * _KC
_NSUB = 16


def _mesh():
    return plsc.VectorSubcoreMesh(core_axis_name="c", subcore_axis_name="s",
                                  num_cores=2, num_subcores=_NSUB)


def _zero_fill(zbuf):
    z16 = jnp.zeros((16,), jnp.float32)

    @pl.loop(0, zbuf.shape[0])
    def _zrow(r):
        for q in range(_HID // 16):
            zbuf[r, pl.ds(q * 16, 16)] = z16


def _scale_chunk(buf, valv, k):
    @pl.loop(0, _CH // 16)
    def _sg(g):
        v16 = valv[k, pl.ds(g * 16, 16)]
        for jj in range(16):
            vb = jnp.broadcast_to(v16[jj], (16,))
            j = g * 16 + jj
            for q in range(_HID // 16):
                buf[j, pl.ds(q * 16, 16)] = buf[j, pl.ds(q * 16, 16)] * vb


def _pipeline_window(table, acc, colv, rowv, valv, bufs, gsems, ssems):
    nb = len(bufs)
    cps = {0: pltpu.async_copy(table.at[colv.at[0]], bufs[0], gsems[0])}
    for k in range(_KC):
        b = k % nb
        cps[k].wait()
        if k + 1 < _KC:
            nxt = (k + 1) % nb
            if k >= 1:
                pltpu.make_async_copy(bufs[nxt], acc.at[rowv.at[k - 1]],
                                      ssems[nxt]).wait()
            cps[k + 1] = pltpu.async_copy(table.at[colv.at[k + 1]],
                                          bufs[nxt], gsems[nxt])
        _scale_chunk(bufs[b], valv, k)
        pltpu.async_copy(bufs[b], acc.at[rowv.at[k]], ssems[b], add=True)
    for k in range(_KC - nb, _KC):
        pltpu.make_async_copy(bufs[k % nb], acc.at[rowv.at[k]],
                              ssems[k % nb]).wait()


def _make_spmm_a(n_dst, nnz_pad):
    per_core_rows = nnz_pad // (2 * _CH)
    per_tile_rows = per_core_rows // _NSUB
    nwin = per_tile_rows // _KC
    rpt = 624
    nzc = rpt // 16

    def body(table, rows2, cols2, vals2, out,
             acc, colv, rowv, valv, b0, b1, zbuf,
             g0, g1, s0, s1):
        c = lax.axis_index("c")
        s = lax.axis_index("s")
        bufs = (b0, b1)
        gsems = (g0, g1)
        ssems = (s0, s1)

        _zero_fill(zbuf)

        @pl.loop(0, nzc)
        def _za(t):
            pltpu.sync_copy(zbuf, acc.at[pl.ds(s * rpt + t * 16, 16), :])

        @pl.when(s == _NSUB - 1)
        def _zrem():
            pltpu.sync_copy(zbuf, acc.at[pl.ds(n_dst - 16, 16), :])

        plsc.subcore_barrier()

        @pl.loop(0, nwin)
        def _win(w):
            crow = pl.multiple_of(
                c * per_core_rows + s * per_tile_rows + w * _KC, 8)
            pltpu.sync_copy(cols2.at[pl.ds(crow, _KC)], colv)
            pltpu.sync_copy(rows2.at[pl.ds(crow, _KC)], rowv)
            pltpu.sync_copy(vals2.at[pl.ds(crow, _KC)], valv)
            _pipeline_window(table, acc, colv, rowv, valv, bufs, gsems, ssems)

        plsc.subcore_barrier()
        base = pl.multiple_of(s * rpt, 8)
        pltpu.sync_copy(acc.at[pl.ds(base, rpt), :],
                        out.at[c].at[pl.ds(base, rpt), :])

        @pl.when(s == _NSUB - 1)
        def _orem():
            pltpu.sync_copy(acc.at[pl.ds(n_dst - 16, 16), :],
                            out.at[c].at[pl.ds(n_dst - 16, 16), :])

    return pl.kernel(
        body,
        out_type=jax.ShapeDtypeStruct((2, n_dst, _HID), jnp.float32),
        mesh=_mesh(),
        scratch_types=[
            pltpu.VMEM_SHARED((n_dst, _HID), jnp.float32),
            pltpu.VMEM((_KC, _CH), jnp.int32),
            pltpu.VMEM((_KC, _CH), jnp.int32),
            pltpu.VMEM((_KC, _CH), jnp.float32),
            pltpu.VMEM((_CH, _HID), jnp.float32),
            pltpu.VMEM((_CH, _HID), jnp.float32),
            pltpu.VMEM((16, _HID), jnp.float32),
            pltpu.SemaphoreType.DMA, pltpu.SemaphoreType.DMA,
            pltpu.SemaphoreType.DMA, pltpu.SemaphoreType.DMA,
        ],
    )


def _make_spmm_x(d_half, nnz_pad):
    nwin_total = nnz_pad // _WIN
    acc_rows = d_half + 2 * _CH
    rpt = acc_rows // _NSUB
    assert rpt % 16 == 0
    rpt_out = d_half // _NSUB

    def body(table, rows2, cols2, vals2, bounds, out,
             acc, colv, rowv, valv, b0, b1, zbuf, wsm,
             g0, g1, s0, s1):
        c = lax.axis_index("c")
        s = lax.axis_index("s")
        bufs = (b0, b1)
        gsems = (g0, g1)
        ssems = (s0, s1)

        _zero_fill(zbuf)

        @pl.loop(0, rpt // 16)
        def _za(t):
            pltpu.sync_copy(zbuf, acc.at[pl.ds(s * rpt + t * 16, 16), :])

        pltpu.sync_copy(bounds.at[c], wsm)
        plsc.subcore_barrier()

        wvec = wsm[...]
        wlo = wvec[0]
        whi = wvec[1]
        w0 = wlo + s
        nsteps = jnp.maximum((whi - w0 + (_NSUB - 1)) // _NSUB, 0)
        roff = c * d_half

        def step(i, carry):
            w = w0 + i * _NSUB
            crow = pl.multiple_of(w * _KC, 8)
            pltpu.sync_copy(cols2.at[pl.ds(crow, _KC)], colv)
            pltpu.sync_copy(rows2.at[pl.ds(crow, _KC)], rowv)
            pltpu.sync_copy(vals2.at[pl.ds(crow, _KC)], valv)

            @pl.loop(0, _KC)
            def _remap(k):
                @pl.loop(0, _CH // 16)
                def _rg(g):
                    r16 = rowv[k, pl.ds(g * 16, 16)]
                    loc = r16 - roff
                    ok = (loc >= 0) & (loc < d_half)
                    rowv[k, pl.ds(g * 16, 16)] = jnp.where(ok, loc, d_half)

            _pipeline_window(table, acc, colv, rowv, valv, bufs, gsems, ssems)
            return carry

        lax.fori_loop(0, nsteps, step, 0)

        plsc.subcore_barrier()
        base = pl.multiple_of(s * rpt_out, 8)
        pltpu.sync_copy(acc.at[pl.ds(base, rpt_out), :],
                        out.at[c].at[pl.ds(base, rpt_out), :])

    return pl.kernel(
        body,
        out_type=jax.ShapeDtypeStruct((2, d_half, _HID), jnp.float32),
        mesh=_mesh(),
        scratch_types=[
            pltpu.VMEM_SHARED((d_half + 2 * _CH, _HID), jnp.float32),
            pltpu.VMEM((_KC, _CH), jnp.int32),
            pltpu.VMEM((_KC, _CH), jnp.int32),
            pltpu.VMEM((_KC, _CH), jnp.float32),
            pltpu.VMEM((_CH, _HID), jnp.float32),
            pltpu.VMEM((_CH, _HID), jnp.float32),
            pltpu.VMEM((16, _HID), jnp.float32),
            pltpu.VMEM((16,), jnp.int32),
            pltpu.SemaphoreType.DMA, pltpu.SemaphoreType.DMA,
            pltpu.SemaphoreType.DMA, pltpu.SemaphoreType.DMA,
        ],
    )


def _dotT(x, w):
    return lax.dot_general(x, w, (((1,), (1,)), ((), ())),
                           preferred_element_type=jnp.float32)


def _mm_relu(acc2, w, blk):
    n = acc2.shape[1]

    def body(a_ref, w_ref, o_ref):
        x = a_ref[0] + a_ref[1]
        o_ref[...] = jnp.maximum(_dotT(x, w_ref[...]), 0.0)

    return pl.pallas_call(
        body,
        grid=(n // blk,),
        in_specs=[pl.BlockSpec((2, blk, _HID), lambda i: (0, i, 0)),
                  pl.BlockSpec((_HID, _HID), lambda i: (0, 0))],
        out_specs=pl.BlockSpec((blk, _HID), lambda i: (i, 0)),
        out_shape=jax.ShapeDtypeStruct((n, _HID), jnp.float32),
    )(acc2, w)


def _post(acc2, w3, emb, ln_g, ln_b, blk):
    n = acc2.shape[1]

    def body(a_ref, w_ref, e_ref, g_ref, b_ref, o_ref):
        x = a_ref[0] + a_ref[1]
        h = jnp.maximum(_dotT(x, w_ref[...]), 0.0)
        e = e_ref[...]
        hr = (1.0 - 0.7) * e + 0.7 * h
        m = jnp.mean(hr, axis=-1, keepdims=True)
        v = jnp.mean((hr - m) ** 2, axis=-1, keepdims=True)
        wh = (hr - m) * lax.rsqrt(v + 1e-5) * g_ref[...] + b_ref[...]
        o_ref[...] = wh + e

    return pl.pallas_call(
        body,
        grid=(n // blk,),
        in_specs=[pl.BlockSpec((2, blk, _HID), lambda i: (0, i, 0)),
                  pl.BlockSpec((_HID, _HID), lambda i: (0, 0)),
                  pl.BlockSpec((blk, _HID), lambda i: (i, 0)),
                  pl.BlockSpec((1, _HID), lambda i: (0, 0)),
                  pl.BlockSpec((1, _HID), lambda i: (0, 0))],
        out_specs=pl.BlockSpec((blk, _HID), lambda i: (i, 0)),
        out_shape=jax.ShapeDtypeStruct((n, _HID), jnp.float32),
    )(acc2, w3, emb, ln_g, ln_b)


def _mlp(doc, wm1, bm1, g1, b1, m1, v1, wm2, bm2, g2, b2, m2, v2, wc, bc,
         blk):
    dn = doc.shape[0]
    hh = _HID // 2

    def body(x_ref, wm1r, bm1r, g1r, b1r, m1r, v1r,
             wm2r, bm2r, g2r, b2r, m2r, v2r, wcr, bcr, o_ref):
        x = x_ref[...]
        z = _dotT(x, wm1r[...]) + bm1r[...]
        t = jnp.maximum((z - m1r[...]) * lax.rsqrt(v1r[...] + 1e-5)
                        * g1r[...] + b1r[...], 0.0)
        z2 = _dotT(t, wm2r[...]) + bm2r[...]
        t2 = jnp.maximum((z2 - m2r[...]) * lax.rsqrt(v2r[...] + 1e-5)
                         * g2r[...] + b2r[...], 0.0)
        o_ref[...] = _dotT(t2, wcr[...]) + bcr[...]

    full = lambda shape: pl.BlockSpec(shape, lambda i: tuple(0 for _ in shape))
    return pl.pallas_call(
        body,
        grid=(dn // blk,),
        in_specs=[pl.BlockSpec((blk, _HID), lambda i: (i, 0)),
                  full((_HID, _HID)), full((1, _HID)), full((1, _HID)),
                  full((1, _HID)), full((1, _HID)), full((1, _HID)),
                  full((hh, _HID)), full((1, hh)), full((1, hh)),
                  full((1, hh)), full((1, hh)), full((1, hh)),
                  full((2, hh)), full((1, 2))],
        out_specs=pl.BlockSpec((blk, 2), lambda i: (i, 0)),
        out_shape=jax.ShapeDtypeStruct((dn, 2), jnp.float32),
    )(doc, wm1, bm1, g1, b1, m1, v1, wm2, bm2, g2, b2, m2, v2, wc, bc)


def kernel(a_row, a_col, a_val, x_row, x_col, x_val, emb, W1, W2, W3,
           ln_g, ln_b, Wm1, bm1, bn1_g, bn1_b, bn1_m, bn1_v,
           Wm2, bm2, bn2_g, bn2_b, bn2_m, bn2_v, Wc, bc):
    i32 = jnp.int32
    n, _ = emb.shape
    e = a_row.shape[0]
    nnzx = x_row.shape[0]
    d = 16384
    dh = d // 2

    unit = 2 * _NSUB * _WIN
    e_pad = ((e + unit - 1) // unit) * unit
    pe = e_pad - e
    zi = jnp.zeros((pe,), i32)
    ar2 = jnp.concatenate([a_row.astype(i32), zi]).reshape(e_pad // _CH, _CH)
    ac2 = jnp.concatenate([a_col.astype(i32), zi]).reshape(e_pad // _CH, _CH)
    av2 = jnp.concatenate([a_val, jnp.zeros((pe,), jnp.float32)]
                          ).reshape(e_pad // _CH, _CH)

    x_pad = ((nnzx + _WIN - 1) // _WIN) * _WIN
    px = x_pad - nnzx
    xr = jnp.concatenate([x_row.astype(i32), jnp.full((px,), d - 1, i32)])
    xr2 = xr.reshape(x_pad // _CH, _CH)
    xc2 = jnp.concatenate([x_col.astype(i32), jnp.zeros((px,), i32)]
                          ).reshape(x_pad // _CH, _CH)
    xv2 = jnp.concatenate([x_val, jnp.zeros((px,), jnp.float32)]
                          ).reshape(x_pad // _CH, _CH)

    p = jnp.searchsorted(xr, dh).astype(i32)
    nw = x_pad // _WIN
    bounds = jnp.zeros((2, 16), i32)
    bounds = bounds.at[0, 1].set((p + _WIN - 1) // _WIN)
    bounds = bounds.at[1, 0].set(p // _WIN)
    bounds = bounds.at[1, 1].set(jnp.int32(nw))

    spmm_a = _make_spmm_a(n, e_pad)
    spmm_x = _make_spmm_x(dh, x_pad)

    acc = spmm_a(emb, ar2, ac2, av2)
    h = _mm_relu(acc, W1, 2000)
    acc = spmm_a(h, ar2, ac2, av2)
    h = _mm_relu(acc, W2, 2000)
    acc = spmm_a(h, ar2, ac2, av2)
    s = _post(acc, W3, emb, ln_g.reshape(1, -1), ln_b.reshape(1, -1), 2000)

    dacc = spmm_x(s, xr2, xc2, xv2, bounds)
    doc = dacc.reshape(d, _HID)
    return _mlp(doc, Wm1, bm1.reshape(1, -1), bn1_g.reshape(1, -1),
                bn1_b.reshape(1, -1), bn1_m.reshape(1, -1),
                bn1_v.reshape(1, -1), Wm2, bm2.reshape(1, -1),
                bn2_g.reshape(1, -1), bn2_b.reshape(1, -1),
                bn2_m.reshape(1, -1), bn2_v.reshape(1, -1),
                Wc, bc.reshape(1, -1), 2048)


# speedup vs baseline: 15.7613x; 1.9162x over previous
import jax
import jax.numpy as jnp
from jax import lax
from jax.experimental import pallas as pl
from jax.experimental.pallas import tpu as pltpu
from jax.experimental.pallas import tpu_sc as plsc

_HID = 128
_CH = 128
_KC = 8
_WIN = _CH * _KC
_NSUB = 16


def _mesh():
    return plsc.VectorSubcoreMesh(core_axis_name="c", subcore_axis_name="s",
                                  num_cores=2, num_subcores=_NSUB)


def _zero_fill(zbuf):
    z16 = jnp.zeros((16,), jnp.float32)

    @pl.loop(0, zbuf.shape[0])
    def _zrow(r):
        for q in range(_HID // 16):
            zbuf[r, pl.ds(q * 16, 16)] = z16


def _scale_chunk(buf, valv, k):
    @pl.loop(0, _CH // 16)
    def _sg(g):
        v16 = valv[k, pl.ds(g * 16, 16)]
        for jj in range(16):
            vb = jnp.broadcast_to(v16[jj], (16,))
            j = g * 16 + jj
            for q in range(_HID // 16):
                buf[j, pl.ds(q * 16, 16)] = buf[j, pl.ds(q * 16, 16)] * vb


def _pipeline_window(table, acc, colv, rowv, valv, bufs, gsems, ssems):
    nb = len(bufs)
    cps = {0: pltpu.async_copy(table.at[colv.at[0]], bufs[0], gsems[0])}
    for k in range(_KC):
        b = k % nb
        cps[k].wait()
        if k + 1 < _KC:
            nxt = (k + 1) % nb
            if k >= 1:
                pltpu.make_async_copy(bufs[nxt], acc.at[rowv.at[k - 1]],
                                      ssems[nxt]).wait()
            cps[k + 1] = pltpu.async_copy(table.at[colv.at[k + 1]],
                                          bufs[nxt], gsems[nxt])
        _scale_chunk(bufs[b], valv, k)
        pltpu.async_copy(bufs[b], acc.at[rowv.at[k]], ssems[b], add=True)
    for k in range(_KC - nb, _KC):
        pltpu.make_async_copy(bufs[k % nb], acc.at[rowv.at[k]],
                              ssems[k % nb]).wait()


def _make_spmm_a(n_dst, nnz_pad):
    per_core_rows = nnz_pad // (2 * _CH)
    per_tile_rows = per_core_rows // _NSUB
    nwin = per_tile_rows // _KC
    rpt = 624
    nzc = rpt // 16

    def body(table, rows2, cols2, vals2, out,
             acc, colv, rowv, valv, b0, b1, zbuf,
             g0, g1, s0, s1):
        c = lax.axis_index("c")
        s = lax.axis_index("s")
        bufs = (b0, b1)
        gsems = (g0, g1)
        ssems = (s0, s1)

        _zero_fill(zbuf)

        @pl.loop(0, nzc)
        def _za(t):
            pltpu.sync_copy(zbuf, acc.at[pl.ds(s * rpt + t * 16, 16), :])

        @pl.when(s == _NSUB - 1)
        def _zrem():
            pltpu.sync_copy(zbuf, acc.at[pl.ds(n_dst - 16, 16), :])

        plsc.subcore_barrier()

        @pl.loop(0, nwin)
        def _win(w):
            crow = pl.multiple_of(
                c * per_core_rows + s * per_tile_rows + w * _KC, 8)
            pltpu.sync_copy(cols2.at[pl.ds(crow, _KC)], colv)
            pltpu.sync_copy(rows2.at[pl.ds(crow, _KC)], rowv)
            pltpu.sync_copy(vals2.at[pl.ds(crow, _KC)], valv)
            _pipeline_window(table, acc, colv, rowv, valv, bufs, gsems, ssems)

        plsc.subcore_barrier()
        base = pl.multiple_of(s * rpt, 8)
        pltpu.sync_copy(acc.at[pl.ds(base, rpt), :],
                        out.at[c].at[pl.ds(base, rpt), :])

        @pl.when(s == _NSUB - 1)
        def _orem():
            pltpu.sync_copy(acc.at[pl.ds(n_dst - 16, 16), :],
                            out.at[c].at[pl.ds(n_dst - 16, 16), :])

    return pl.kernel(
        body,
        out_type=jax.ShapeDtypeStruct((2, n_dst, _HID), jnp.float32),
        mesh=_mesh(),
        scratch_types=[
            pltpu.VMEM_SHARED((n_dst, _HID), jnp.float32),
            pltpu.VMEM((_KC, _CH), jnp.int32),
            pltpu.VMEM((_KC, _CH), jnp.int32),
            pltpu.VMEM((_KC, _CH), jnp.float32),
            pltpu.VMEM((_CH, _HID), jnp.float32),
            pltpu.VMEM((_CH, _HID), jnp.float32),
            pltpu.VMEM((16, _HID), jnp.float32),
            pltpu.SemaphoreType.DMA, pltpu.SemaphoreType.DMA,
            pltpu.SemaphoreType.DMA, pltpu.SemaphoreType.DMA,
        ],
    )


def _make_spmm_x(d_half, nnz_pad):
    nwin_total = nnz_pad // _WIN
    acc_rows = d_half + 2 * _CH
    rpt = acc_rows // _NSUB
    assert rpt % 16 == 0
    rpt_out = d_half // _NSUB

    def body(table, rows2, cols2, vals2, bounds, out,
             acc, colv, rowv, valv, b0, b1, zbuf, wsm,
             g0, g1, s0, s1):
        c = lax.axis_index("c")
        s = lax.axis_index("s")
        bufs = (b0, b1)
        gsems = (g0, g1)
        ssems = (s0, s1)

        _zero_fill(zbuf)

        @pl.loop(0, rpt // 16)
        def _za(t):
            pltpu.sync_copy(zbuf, acc.at[pl.ds(s * rpt + t * 16, 16), :])

        pltpu.sync_copy(bounds.at[c], wsm)
        plsc.subcore_barrier()

        wvec = wsm[...]
        wlo = wvec[0]
        whi = wvec[1]
        w0 = wlo + s
        nsteps = jnp.maximum((whi - w0 + (_NSUB - 1)) // _NSUB, 0)
        roff = c * d_half

        def step(i, carry):
            w = w0 + i * _NSUB
            crow = pl.multiple_of(w * _KC, 8)
            pltpu.sync_copy(cols2.at[pl.ds(crow, _KC)], colv)
            pltpu.sync_copy(rows2.at[pl.ds(crow, _KC)], rowv)
            pltpu.sync_copy(vals2.at[pl.ds(crow, _KC)], valv)

            @pl.loop(0, _KC)
            def _remap(k):
                @pl.loop(0, _CH // 16)
                def _rg(g):
                    r16 = rowv[k, pl.ds(g * 16, 16)]
                    loc = r16 - roff
                    ok = (loc >= 0) & (loc < d_half)
                    rowv[k, pl.ds(g * 16, 16)] = jnp.where(ok, loc, d_half)

            _pipeline_window(table, acc, colv, rowv, valv, bufs, gsems, ssems)
            return carry

        lax.fori_loop(0, nsteps, step, 0)

        plsc.subcore_barrier()
        base = pl.multiple_of(s * rpt_out, 8)
        pltpu.sync_copy(acc.at[pl.ds(base, rpt_out), :],
                        out.at[c].at[pl.ds(base, rpt_out), :])

    return pl.kernel(
        body,
        out_type=jax.ShapeDtypeStruct((2, d_half, _HID), jnp.float32),
        mesh=_mesh(),
        scratch_types=[
            pltpu.VMEM_SHARED((d_half + 2 * _CH, _HID), jnp.float32),
            pltpu.VMEM((_KC, _CH), jnp.int32),
            pltpu.VMEM((_KC, _CH), jnp.int32),
            pltpu.VMEM((_KC, _CH), jnp.float32),
            pltpu.VMEM((_CH, _HID), jnp.float32),
            pltpu.VMEM((_CH, _HID), jnp.float32),
            pltpu.VMEM((16, _HID), jnp.float32),
            pltpu.VMEM((16,), jnp.int32),
            pltpu.SemaphoreType.DMA, pltpu.SemaphoreType.DMA,
            pltpu.SemaphoreType.DMA, pltpu.SemaphoreType.DMA,
        ],
    )


def _dotT(x, w):
    return lax.dot_general(x, w, (((1,), (1,)), ((), ())),
                           preferred_element_type=jnp.float32)


def _mm_relu(acc2, w, blk):
    n = acc2.shape[1]

    def body(a_ref, w_ref, o_ref):
        x = a_ref[0] + a_ref[1]
        o_ref[...] = jnp.maximum(_dotT(x, w_ref[...]), 0.0)

    return pl.pallas_call(
        body,
        grid=(n // blk,),
        in_specs=[pl.BlockSpec((2, blk, _HID), lambda i: (0, i, 0)),
                  pl.BlockSpec((_HID, _HID), lambda i: (0, 0))],
        out_specs=pl.BlockSpec((blk, _HID), lambda i: (i, 0)),
        out_shape=jax.ShapeDtypeStruct((n, _HID), jnp.float32),
    )(acc2, w)


def _post(acc2, w3, emb, ln_g, ln_b, blk):
    n = acc2.shape[1]

    def body(a_ref, w_ref, e_ref, g_ref, b_ref, o_ref):
        x = a_ref[0] + a_ref[1]
        h = jnp.maximum(_dotT(x, w_ref[...]), 0.0)
        e = e_ref[...]
        hr = (1.0 - 0.7) * e + 0.7 * h
        m = jnp.mean(hr, axis=-1, keepdims=True)
        v = jnp.mean((hr - m) ** 2, axis=-1, keepdims=True)
        wh = (hr - m) * lax.rsqrt(v + 1e-5) * g_ref[...] + b_ref[...]
        o_ref[...] = wh + e

    return pl.pallas_call(
        body,
        grid=(n // blk,),
        in_specs=[pl.BlockSpec((2, blk, _HID), lambda i: (0, i, 0)),
                  pl.BlockSpec((_HID, _HID), lambda i: (0, 0)),
                  pl.BlockSpec((blk, _HID), lambda i: (i, 0)),
                  pl.BlockSpec((1, _HID), lambda i: (0, 0)),
                  pl.BlockSpec((1, _HID), lambda i: (0, 0))],
        out_specs=pl.BlockSpec((blk, _HID), lambda i: (i, 0)),
        out_shape=jax.ShapeDtypeStruct((n, _HID), jnp.float32),
    )(acc2, w3, emb, ln_g, ln_b)


def _mlp(doc, wm1, bm1, g1, b1, m1, v1, wm2, bm2, g2, b2, m2, v2, wc, bc,
         blk):
    dn = doc.shape[0]
    hh = _HID // 2

    def body(x_ref, wm1r, bm1r, g1r, b1r, m1r, v1r,
             wm2r, bm2r, g2r, b2r, m2r, v2r, wcr, bcr, o_ref):
        x = x_ref[...]
        z = _dotT(x, wm1r[...]) + bm1r[...]
        t = jnp.maximum((z - m1r[...]) * lax.rsqrt(v1r[...] + 1e-5)
                        * g1r[...] + b1r[...], 0.0)
        z2 = _dotT(t, wm2r[...]) + bm2r[...]
        t2 = jnp.maximum((z2 - m2r[...]) * lax.rsqrt(v2r[...] + 1e-5)
                         * g2r[...] + b2r[...], 0.0)
        o_ref[...] = _dotT(t2, wcr[...]) + bcr[...]

    full = lambda shape: pl.BlockSpec(shape, lambda i: tuple(0 for _ in shape))
    return pl.pallas_call(
        body,
        grid=(dn // blk,),
        in_specs=[pl.BlockSpec((blk, _HID), lambda i: (i, 0)),
                  full((_HID, _HID)), full((1, _HID)), full((1, _HID)),
                  full((1, _HID)), full((1, _HID)), full((1, _HID)),
                  full((hh, _HID)), full((1, hh)), full((1, hh)),
                  full((1, hh)), full((1, hh)), full((1, hh)),
                  full((2, hh)), full((1, 2))],
        out_specs=pl.BlockSpec((blk, 2), lambda i: (i, 0)),
        out_shape=jax.ShapeDtypeStruct((dn, 2), jnp.float32),
    )(doc, wm1, bm1, g1, b1, m1, v1, wm2, bm2, g2, b2, m2, v2, wc, bc)


def kernel(a_row, a_col, a_val, x_row, x_col, x_val, emb, W1, W2, W3,
           ln_g, ln_b, Wm1, bm1, bn1_g, bn1_b, bn1_m, bn1_v,
           Wm2, bm2, bn2_g, bn2_b, bn2_m, bn2_v, Wc, bc):
    i32 = jnp.int32
    n, _ = emb.shape
    e = a_row.shape[0]
    nnzx = x_row.shape[0]
    d = 16384
    dh = d // 2

    unit = 2 * _NSUB * _WIN
    e_pad = ((e + unit - 1) // unit) * unit
    pe = e_pad - e
    zi = jnp.arange(pe, dtype=i32) % jnp.int32(n)
    ar2 = jnp.concatenate([a_row.astype(i32), zi]).reshape(e_pad // _CH, _CH)
    ac2 = jnp.concatenate([a_col.astype(i32), zi]).reshape(e_pad // _CH, _CH)
    av2 = jnp.concatenate([a_val, jnp.zeros((pe,), jnp.float32)]
                          ).reshape(e_pad // _CH, _CH)

    x_pad = ((nnzx + _WIN - 1) // _WIN) * _WIN
    px = x_pad - nnzx
    xr = jnp.concatenate([x_row.astype(i32), jnp.full((px,), d - 1, i32)])
    xr2 = xr.reshape(x_pad // _CH, _CH)
    xc2 = jnp.concatenate([x_col.astype(i32), jnp.zeros((px,), i32)]
                          ).reshape(x_pad // _CH, _CH)
    xv2 = jnp.concatenate([x_val, jnp.zeros((px,), jnp.float32)]
                          ).reshape(x_pad // _CH, _CH)

    p = jnp.searchsorted(xr, dh).astype(i32)
    nw = x_pad // _WIN
    bounds = jnp.zeros((2, 16), i32)
    bounds = bounds.at[0, 1].set((p + _WIN - 1) // _WIN)
    bounds = bounds.at[1, 0].set(p // _WIN)
    bounds = bounds.at[1, 1].set(jnp.int32(nw))

    spmm_a = _make_spmm_a(n, e_pad)
    spmm_x = _make_spmm_x(dh, x_pad)

    acc = spmm_a(emb, ar2, ac2, av2)
    h = _mm_relu(acc, W1, 2000)
    acc = spmm_a(h, ar2, ac2, av2)
    h = _mm_relu(acc, W2, 2000)
    acc = spmm_a(h, ar2, ac2, av2)
    s = _post(acc, W3, emb, ln_g.reshape(1, -1), ln_b.reshape(1, -1), 2000)

    dacc = spmm_x(s, xr2, xc2, xv2, bounds)
    doc = dacc.reshape(d, _HID)
    return _mlp(doc, Wm1, bm1.reshape(1, -1), bn1_g.reshape(1, -1),
                bn1_b.reshape(1, -1), bn1_m.reshape(1, -1),
                bn1_v.reshape(1, -1), Wm2, bm2.reshape(1, -1),
                bn2_g.reshape(1, -1), bn2_b.reshape(1, -1),
                bn2_m.reshape(1, -1), bn2_v.reshape(1, -1),
                Wc, bc.reshape(1, -1), 2048)


# speedup vs baseline: 15.8458x; 1.0054x over previous
import jax
import jax.numpy as jnp
from jax import lax
from jax.experimental import pallas as pl
from jax.experimental.pallas import tpu as pltpu
from jax.experimental.pallas import tpu_sc as plsc

_HID = 128
_CH = 128
_KC = 8
_WIN = _CH * _KC
_NSUB = 16


def _mesh():
    return plsc.VectorSubcoreMesh(core_axis_name="c", subcore_axis_name="s",
                                  num_cores=2, num_subcores=_NSUB)


def _zero_fill(zbuf):
    z16 = jnp.zeros((16,), jnp.float32)

    @pl.loop(0, zbuf.shape[0])
    def _zrow(r):
        for q in range(_HID // 16):
            zbuf[r, pl.ds(q * 16, 16)] = z16


def _scale_chunk(buf, valv, k):
    @pl.loop(0, _CH // 16)
    def _sg(g):
        v16 = valv[k, pl.ds(g * 16, 16)]
        for jj in range(16):
            vb = jnp.broadcast_to(v16[jj], (16,))
            j = g * 16 + jj
            for q in range(_HID // 16):
                buf[j, pl.ds(q * 16, 16)] = buf[j, pl.ds(q * 16, 16)] * vb


def _pipeline_window(table, acc, colv, rowv, valv, bufs, gsems, ssems):
    nb = len(bufs)
    ahead = nb - 1
    cps = {}
    for k in range(min(ahead, _KC)):
        cps[k] = pltpu.async_copy(table.at[colv.at[k]], bufs[k % nb],
                                  gsems[k % nb])
    for k in range(_KC):
        b = k % nb
        cps[k].wait()
        nk = k + ahead
        if nk < _KC:
            nbi = nk % nb
            if nk >= nb:
                pltpu.make_async_copy(bufs[nbi], acc.at[rowv.at[nk - nb]],
                                      ssems[nbi]).wait()
            cps[nk] = pltpu.async_copy(table.at[colv.at[nk]],
                                       bufs[nbi], gsems[nbi])
        _scale_chunk(bufs[b], valv, k)
        pltpu.async_copy(bufs[b], acc.at[rowv.at[k]], ssems[b], add=True)
    for k in range(max(_KC - nb, 0), _KC):
        pltpu.make_async_copy(bufs[k % nb], acc.at[rowv.at[k]],
                              ssems[k % nb]).wait()


def _make_spmm_a(n_dst, nnz_pad):
    per_core_rows = nnz_pad // (2 * _CH)
    per_tile_rows = per_core_rows // _NSUB
    nwin = per_tile_rows // _KC
    rpt = 624
    nzc = rpt // 16

    def body(table, rows2, cols2, vals2, out,
             acc, colv, rowv, valv, b0, b1, zbuf,
             g0, g1, s0, s1):
        c = lax.axis_index("c")
        s = lax.axis_index("s")
        bufs = (b0, b1)
        gsems = (g0, g1)
        ssems = (s0, s1)

        _zero_fill(zbuf)

        @pl.loop(0, nzc)
        def _za(t):
            pltpu.sync_copy(zbuf, acc.at[pl.ds(s * rpt + t * 16, 16), :])

        @pl.when(s == _NSUB - 1)
        def _zrem():
            pltpu.sync_copy(zbuf, acc.at[pl.ds(n_dst - 16, 16), :])

        plsc.subcore_barrier()

        @pl.loop(0, nwin)
        def _win(w):
            crow = pl.multiple_of(
                c * per_core_rows + s * per_tile_rows + w * _KC, 8)
            pltpu.sync_copy(cols2.at[pl.ds(crow, _KC)], colv)
            pltpu.sync_copy(rows2.at[pl.ds(crow, _KC)], rowv)
            pltpu.sync_copy(vals2.at[pl.ds(crow, _KC)], valv)
            _pipeline_window(table, acc, colv, rowv, valv, bufs, gsems, ssems)

        plsc.subcore_barrier()
        base = pl.multiple_of(s * rpt, 8)
        pltpu.sync_copy(acc.at[pl.ds(base, rpt), :],
                        out.at[c].at[pl.ds(base, rpt), :])

        @pl.when(s == _NSUB - 1)
        def _orem():
            pltpu.sync_copy(acc.at[pl.ds(n_dst - 16, 16), :],
                            out.at[c].at[pl.ds(n_dst - 16, 16), :])

    return pl.kernel(
        body,
        out_type=jax.ShapeDtypeStruct((2, n_dst, _HID), jnp.float32),
        mesh=_mesh(),
        scratch_types=[
            pltpu.VMEM_SHARED((n_dst, _HID), jnp.float32),
            pltpu.VMEM((_KC, _CH), jnp.int32),
            pltpu.VMEM((_KC, _CH), jnp.int32),
            pltpu.VMEM((_KC, _CH), jnp.float32),
            pltpu.VMEM((_CH, _HID), jnp.float32),
            pltpu.VMEM((_CH, _HID), jnp.float32),
            pltpu.VMEM((16, _HID), jnp.float32),
            pltpu.SemaphoreType.DMA, pltpu.SemaphoreType.DMA,
            pltpu.SemaphoreType.DMA, pltpu.SemaphoreType.DMA,
        ],
    )


def _make_spmm_x(d_half, nnz_pad):
    nwin_total = nnz_pad // _WIN
    acc_rows = d_half + 2 * _CH
    rpt = acc_rows // _NSUB
    assert rpt % 16 == 0
    rpt_out = d_half // _NSUB

    def body(table, rows2, cols2, vals2, bounds, out,
             acc, colv, rowv, valv, b0, b1, b2, zbuf, wsm,
             g0, g1, g2, s0, s1, s2):
        c = lax.axis_index("c")
        s = lax.axis_index("s")
        bufs = (b0, b1, b2)
        gsems = (g0, g1, g2)
        ssems = (s0, s1, s2)

        _zero_fill(zbuf)

        @pl.loop(0, rpt // 16)
        def _za(t):
            pltpu.sync_copy(zbuf, acc.at[pl.ds(s * rpt + t * 16, 16), :])

        pltpu.sync_copy(bounds.at[c], wsm)
        plsc.subcore_barrier()

        wvec = wsm[...]
        wlo = wvec[0]
        whi = wvec[1]
        w0 = wlo + s
        nsteps = jnp.maximum((whi - w0 + (_NSUB - 1)) // _NSUB, 0)
        roff = c * d_half

        def step(i, carry):
            w = w0 + i * _NSUB
            crow = pl.multiple_of(w * _KC, 8)
            pltpu.sync_copy(cols2.at[pl.ds(crow, _KC)], colv)
            pltpu.sync_copy(rows2.at[pl.ds(crow, _KC)], rowv)
            pltpu.sync_copy(vals2.at[pl.ds(crow, _KC)], valv)

            @pl.loop(0, _KC)
            def _remap(k):
                @pl.loop(0, _CH // 16)
                def _rg(g):
                    r16 = rowv[k, pl.ds(g * 16, 16)]
                    loc = r16 - roff
                    ok = (loc >= 0) & (loc < d_half)
                    rowv[k, pl.ds(g * 16, 16)] = jnp.where(ok, loc, d_half)

            _pipeline_window(table, acc, colv, rowv, valv, bufs, gsems, ssems)
            return carry

        lax.fori_loop(0, nsteps, step, 0)

        plsc.subcore_barrier()
        base = pl.multiple_of(s * rpt_out, 8)
        pltpu.sync_copy(acc.at[pl.ds(base, rpt_out), :],
                        out.at[c].at[pl.ds(base, rpt_out), :])

    return pl.kernel(
        body,
        out_type=jax.ShapeDtypeStruct((2, d_half, _HID), jnp.float32),
        mesh=_mesh(),
        scratch_types=[
            pltpu.VMEM_SHARED((d_half + 2 * _CH, _HID), jnp.float32),
            pltpu.VMEM((_KC, _CH), jnp.int32),
            pltpu.VMEM((_KC, _CH), jnp.int32),
            pltpu.VMEM((_KC, _CH), jnp.float32),
            pltpu.VMEM((_CH, _HID), jnp.float32),
            pltpu.VMEM((_CH, _HID), jnp.float32),
            pltpu.VMEM((_CH, _HID), jnp.float32),
            pltpu.VMEM((16, _HID), jnp.float32),
            pltpu.VMEM((16,), jnp.int32),
            pltpu.SemaphoreType.DMA, pltpu.SemaphoreType.DMA,
            pltpu.SemaphoreType.DMA, pltpu.SemaphoreType.DMA,
            pltpu.SemaphoreType.DMA, pltpu.SemaphoreType.DMA,
        ],
    )


def _dotT(x, w):
    return lax.dot_general(x, w, (((1,), (1,)), ((), ())),
                           preferred_element_type=jnp.float32)


def _mm_relu(acc2, w, blk):
    n = acc2.shape[1]

    def body(a_ref, w_ref, o_ref):
        x = a_ref[0] + a_ref[1]
        o_ref[...] = jnp.maximum(_dotT(x, w_ref[...]), 0.0)

    return pl.pallas_call(
        body,
        grid=(n // blk,),
        in_specs=[pl.BlockSpec((2, blk, _HID), lambda i: (0, i, 0)),
                  pl.BlockSpec((_HID, _HID), lambda i: (0, 0))],
        out_specs=pl.BlockSpec((blk, _HID), lambda i: (i, 0)),
        out_shape=jax.ShapeDtypeStruct((n, _HID), jnp.float32),
    )(acc2, w)


def _post(acc2, w3, emb, ln_g, ln_b, blk):
    n = acc2.shape[1]

    def body(a_ref, w_ref, e_ref, g_ref, b_ref, o_ref):
        x = a_ref[0] + a_ref[1]
        h = jnp.maximum(_dotT(x, w_ref[...]), 0.0)
        e = e_ref[...]
        hr = (1.0 - 0.7) * e + 0.7 * h
        m = jnp.mean(hr, axis=-1, keepdims=True)
        v = jnp.mean((hr - m) ** 2, axis=-1, keepdims=True)
        wh = (hr - m) * lax.rsqrt(v + 1e-5) * g_ref[...] + b_ref[...]
        o_ref[...] = wh + e

    return pl.pallas_call(
        body,
        grid=(n // blk,),
        in_specs=[pl.BlockSpec((2, blk, _HID), lambda i: (0, i, 0)),
                  pl.BlockSpec((_HID, _HID), lambda i: (0, 0)),
                  pl.BlockSpec((blk, _HID), lambda i: (i, 0)),
                  pl.BlockSpec((1, _HID), lambda i: (0, 0)),
                  pl.BlockSpec((1, _HID), lambda i: (0, 0))],
        out_specs=pl.BlockSpec((blk, _HID), lambda i: (i, 0)),
        out_shape=jax.ShapeDtypeStruct((n, _HID), jnp.float32),
    )(acc2, w3, emb, ln_g, ln_b)


def _mlp(doc, wm1, bm1, g1, b1, m1, v1, wm2, bm2, g2, b2, m2, v2, wc, bc,
         blk):
    dn = doc.shape[0]
    hh = _HID // 2

    def body(x_ref, wm1r, bm1r, g1r, b1r, m1r, v1r,
             wm2r, bm2r, g2r, b2r, m2r, v2r, wcr, bcr, o_ref):
        x = x_ref[...]
        z = _dotT(x, wm1r[...]) + bm1r[...]
        t = jnp.maximum((z - m1r[...]) * lax.rsqrt(v1r[...] + 1e-5)
                        * g1r[...] + b1r[...], 0.0)
        z2 = _dotT(t, wm2r[...]) + bm2r[...]
        t2 = jnp.maximum((z2 - m2r[...]) * lax.rsqrt(v2r[...] + 1e-5)
                         * g2r[...] + b2r[...], 0.0)
        o_ref[...] = _dotT(t2, wcr[...]) + bcr[...]

    full = lambda shape: pl.BlockSpec(shape, lambda i: tuple(0 for _ in shape))
    return pl.pallas_call(
        body,
        grid=(dn // blk,),
        in_specs=[pl.BlockSpec((blk, _HID), lambda i: (i, 0)),
                  full((_HID, _HID)), full((1, _HID)), full((1, _HID)),
                  full((1, _HID)), full((1, _HID)), full((1, _HID)),
                  full((hh, _HID)), full((1, hh)), full((1, hh)),
                  full((1, hh)), full((1, hh)), full((1, hh)),
                  full((2, hh)), full((1, 2))],
        out_specs=pl.BlockSpec((blk, 2), lambda i: (i, 0)),
        out_shape=jax.ShapeDtypeStruct((dn, 2), jnp.float32),
    )(doc, wm1, bm1, g1, b1, m1, v1, wm2, bm2, g2, b2, m2, v2, wc, bc)


def kernel(a_row, a_col, a_val, x_row, x_col, x_val, emb, W1, W2, W3,
           ln_g, ln_b, Wm1, bm1, bn1_g, bn1_b, bn1_m, bn1_v,
           Wm2, bm2, bn2_g, bn2_b, bn2_m, bn2_v, Wc, bc):
    i32 = jnp.int32
    n, _ = emb.shape
    e = a_row.shape[0]
    nnzx = x_row.shape[0]
    d = 16384
    dh = d // 2

    unit = 2 * _NSUB * _WIN
    e_pad = ((e + unit - 1) // unit) * unit
    pe = e_pad - e
    zi = jnp.arange(pe, dtype=i32) % jnp.int32(n)
    ar2 = jnp.concatenate([a_row.astype(i32), zi]).reshape(e_pad // _CH, _CH)
    ac2 = jnp.concatenate([a_col.astype(i32), zi]).reshape(e_pad // _CH, _CH)
    av2 = jnp.concatenate([a_val, jnp.zeros((pe,), jnp.float32)]
                          ).reshape(e_pad // _CH, _CH)

    x_pad = ((nnzx + _WIN - 1) // _WIN) * _WIN
    px = x_pad - nnzx
    xr = jnp.concatenate([x_row.astype(i32), jnp.full((px,), d - 1, i32)])
    xr2 = xr.reshape(x_pad // _CH, _CH)
    xc2 = jnp.concatenate([x_col.astype(i32), jnp.zeros((px,), i32)]
                          ).reshape(x_pad // _CH, _CH)
    xv2 = jnp.concatenate([x_val, jnp.zeros((px,), jnp.float32)]
                          ).reshape(x_pad // _CH, _CH)

    p = jnp.searchsorted(xr, dh).astype(i32)
    nw = x_pad // _WIN
    bounds = jnp.zeros((2, 16), i32)
    bounds = bounds.at[0, 1].set((p + _WIN - 1) // _WIN)
    bounds = bounds.at[1, 0].set(p // _WIN)
    bounds = bounds.at[1, 1].set(jnp.int32(nw))

    spmm_a = _make_spmm_a(n, e_pad)
    spmm_x = _make_spmm_x(dh, x_pad)

    acc = spmm_a(emb, ar2, ac2, av2)
    h = _mm_relu(acc, W1, 2000)
    acc = spmm_a(h, ar2, ac2, av2)
    h = _mm_relu(acc, W2, 2000)
    acc = spmm_a(h, ar2, ac2, av2)
    s = _post(acc, W3, emb, ln_g.reshape(1, -1), ln_b.reshape(1, -1), 2000)

    dacc = spmm_x(s, xr2, xc2, xv2, bounds)
    doc = dacc.reshape(d, _HID)
    return _mlp(doc, Wm1, bm1.reshape(1, -1), bn1_g.reshape(1, -1),
                bn1_b.reshape(1, -1), bn1_m.reshape(1, -1),
                bn1_v.reshape(1, -1), Wm2, bm2.reshape(1, -1),
                bn2_g.reshape(1, -1), bn2_b.reshape(1, -1),
                bn2_m.reshape(1, -1), bn2_v.reshape(1, -1),
                Wc, bc.reshape(1, -1), 2048)


# speedup vs baseline: 16.8091x; 1.0608x over previous
import jax
import jax.numpy as jnp
from jax import lax
from jax.experimental import pallas as pl
from jax.experimental.pallas import tpu as pltpu
from jax.experimental.pallas import tpu_sc as plsc

_HID = 128
_CH = 128
_KC = 8
_WIN = _CH * _KC
_NSUB = 16


def _mesh():
    return plsc.VectorSubcoreMesh(core_axis_name="c", subcore_axis_name="s",
                                  num_cores=2, num_subcores=_NSUB)


def _zero_fill(zbuf):
    z16 = jnp.zeros((16,), jnp.float32)

    @pl.loop(0, zbuf.shape[0])
    def _zrow(r):
        for q in range(_HID // 16):
            zbuf[r, pl.ds(q * 16, 16)] = z16


def _scale_chunk(buf, valv, k):
    @pl.loop(0, _CH // 16)
    def _sg(g):
        v16 = valv[k, pl.ds(g * 16, 16)]
        for jj in range(16):
            vb = jnp.broadcast_to(v16[jj], (16,))
            j = g * 16 + jj
            for q in range(_HID // 16):
                buf[j, pl.ds(q * 16, 16)] = buf[j, pl.ds(q * 16, 16)] * vb


def _pipeline_window(table, acc, colv, rowv, valv, bufs, gsems, ssems):
    nb = len(bufs)
    ahead = nb - 1
    cps = {}
    for k in range(min(ahead, _KC)):
        cps[k] = pltpu.async_copy(table.at[colv.at[k]], bufs[k % nb],
                                  gsems[k % nb])
    for k in range(_KC):
        b = k % nb
        cps[k].wait()
        nk = k + ahead
        if nk < _KC:
            nbi = nk % nb
            if nk >= nb:
                pltpu.make_async_copy(bufs[nbi], acc.at[rowv.at[nk - nb]],
                                      ssems[nbi]).wait()
            cps[nk] = pltpu.async_copy(table.at[colv.at[nk]],
                                       bufs[nbi], gsems[nbi])
        _scale_chunk(bufs[b], valv, k)
        pltpu.async_copy(bufs[b], acc.at[rowv.at[k]], ssems[b], add=True)
    for k in range(max(_KC - nb, 0), _KC):
        pltpu.make_async_copy(bufs[k % nb], acc.at[rowv.at[k]],
                              ssems[k % nb]).wait()


def _make_spmm_a(n_dst, nnz_pad):
    per_core_rows = nnz_pad // (2 * _CH)
    per_tile_rows = per_core_rows // _NSUB
    nwin = per_tile_rows // _KC
    rpt = 624
    nzc = rpt // 16

    def body(table, rows2, cols2, vals2, out,
             acc, colv, rowv, valv, b0, b1, zbuf,
             g0, g1, s0, s1):
        c = lax.axis_index("c")
        s = lax.axis_index("s")
        bufs = (b0, b1)
        gsems = (g0, g1)
        ssems = (s0, s1)

        _zero_fill(zbuf)

        @pl.loop(0, nzc)
        def _za(t):
            pltpu.sync_copy(zbuf, acc.at[pl.ds(s * rpt + t * 16, 16), :])

        @pl.when(s == _NSUB - 1)
        def _zrem():
            pltpu.sync_copy(zbuf, acc.at[pl.ds(n_dst - 16, 16), :])

        plsc.subcore_barrier()

        @pl.loop(0, nwin)
        def _win(w):
            crow = pl.multiple_of(
                c * per_core_rows + s * per_tile_rows + w * _KC, 8)
            c1 = pltpu.async_copy(cols2.at[pl.ds(crow, _KC)], colv, g0)
            c2 = pltpu.async_copy(rows2.at[pl.ds(crow, _KC)], rowv, g1)
            c3 = pltpu.async_copy(vals2.at[pl.ds(crow, _KC)], valv, s0)
            c1.wait(); c2.wait(); c3.wait()
            _pipeline_window(table, acc, colv, rowv, valv, bufs, gsems, ssems)

        plsc.subcore_barrier()
        base = pl.multiple_of(s * rpt, 8)
        pltpu.sync_copy(acc.at[pl.ds(base, rpt), :],
                        out.at[c].at[pl.ds(base, rpt), :])

        @pl.when(s == _NSUB - 1)
        def _orem():
            pltpu.sync_copy(acc.at[pl.ds(n_dst - 16, 16), :],
                            out.at[c].at[pl.ds(n_dst - 16, 16), :])

    return pl.kernel(
        body,
        out_type=jax.ShapeDtypeStruct((2, n_dst, _HID), jnp.float32),
        mesh=_mesh(),
        scratch_types=[
            pltpu.VMEM_SHARED((n_dst, _HID), jnp.float32),
            pltpu.VMEM((_KC, _CH), jnp.int32),
            pltpu.VMEM((_KC, _CH), jnp.int32),
            pltpu.VMEM((_KC, _CH), jnp.float32),
            pltpu.VMEM((_CH, _HID), jnp.float32),
            pltpu.VMEM((_CH, _HID), jnp.float32),
            pltpu.VMEM((16, _HID), jnp.float32),
            pltpu.SemaphoreType.DMA, pltpu.SemaphoreType.DMA,
            pltpu.SemaphoreType.DMA, pltpu.SemaphoreType.DMA,
        ],
    )


def _make_spmm_x(d_half, nnz_pad):
    nwin_total = nnz_pad // _WIN
    acc_rows = d_half + 2 * _CH
    rpt = acc_rows // _NSUB
    assert rpt % 16 == 0
    rpt_out = d_half // _NSUB

    def body(table, rows2, cols2, vals2, bounds, out,
             acc, colv, rowv, valv, b0, b1, b2, zbuf, wsm,
             g0, g1, g2, s0, s1, s2):
        c = lax.axis_index("c")
        s = lax.axis_index("s")
        bufs = (b0, b1, b2)
        gsems = (g0, g1, g2)
        ssems = (s0, s1, s2)

        _zero_fill(zbuf)

        @pl.loop(0, rpt // 16)
        def _za(t):
            pltpu.sync_copy(zbuf, acc.at[pl.ds(s * rpt + t * 16, 16), :])

        pltpu.sync_copy(bounds.at[c], wsm)
        plsc.subcore_barrier()

        wvec = wsm[...]
        wlo = wvec[0]
        whi = wvec[1]
        w0 = wlo + s
        nsteps = jnp.maximum((whi - w0 + (_NSUB - 1)) // _NSUB, 0)
        roff = c * d_half

        def step(i, carry):
            w = w0 + i * _NSUB
            crow = pl.multiple_of(w * _KC, 8)
            c1 = pltpu.async_copy(cols2.at[pl.ds(crow, _KC)], colv, g0)
            c2 = pltpu.async_copy(rows2.at[pl.ds(crow, _KC)], rowv, g1)
            c3 = pltpu.async_copy(vals2.at[pl.ds(crow, _KC)], valv, s0)
            c1.wait(); c2.wait(); c3.wait()

            @pl.loop(0, _KC)
            def _remap(k):
                @pl.loop(0, _CH // 16)
                def _rg(g):
                    r16 = rowv[k, pl.ds(g * 16, 16)]
                    loc = r16 - roff
                    ok = (loc >= 0) & (loc < d_half)
                    rowv[k, pl.ds(g * 16, 16)] = jnp.where(ok, loc, d_half)

            _pipeline_window(table, acc, colv, rowv, valv, bufs, gsems, ssems)
            return carry

        lax.fori_loop(0, nsteps, step, 0)

        plsc.subcore_barrier()
        base = pl.multiple_of(s * rpt_out, 8)
        pltpu.sync_copy(acc.at[pl.ds(base, rpt_out), :],
                        out.at[c].at[pl.ds(base, rpt_out), :])

    return pl.kernel(
        body,
        out_type=jax.ShapeDtypeStruct((2, d_half, _HID), jnp.float32),
        mesh=_mesh(),
        scratch_types=[
            pltpu.VMEM_SHARED((d_half + 2 * _CH, _HID), jnp.float32),
            pltpu.VMEM((_KC, _CH), jnp.int32),
            pltpu.VMEM((_KC, _CH), jnp.int32),
            pltpu.VMEM((_KC, _CH), jnp.float32),
            pltpu.VMEM((_CH, _HID), jnp.float32),
            pltpu.VMEM((_CH, _HID), jnp.float32),
            pltpu.VMEM((_CH, _HID), jnp.float32),
            pltpu.VMEM((16, _HID), jnp.float32),
            pltpu.VMEM((16,), jnp.int32),
            pltpu.SemaphoreType.DMA, pltpu.SemaphoreType.DMA,
            pltpu.SemaphoreType.DMA, pltpu.SemaphoreType.DMA,
            pltpu.SemaphoreType.DMA, pltpu.SemaphoreType.DMA,
        ],
    )


def _dotT(x, w):
    return lax.dot_general(x, w, (((1,), (1,)), ((), ())),
                           preferred_element_type=jnp.float32)


def _mm_relu(acc2, w, blk):
    n = acc2.shape[1]

    def body(a_ref, w_ref, o_ref):
        x = a_ref[0] + a_ref[1]
        o_ref[...] = jnp.maximum(_dotT(x, w_ref[...]), 0.0)

    return pl.pallas_call(
        body,
        grid=(n // blk,),
        in_specs=[pl.BlockSpec((2, blk, _HID), lambda i: (0, i, 0)),
                  pl.BlockSpec((_HID, _HID), lambda i: (0, 0))],
        out_specs=pl.BlockSpec((blk, _HID), lambda i: (i, 0)),
        out_shape=jax.ShapeDtypeStruct((n, _HID), jnp.float32),
    )(acc2, w)


def _post(acc2, w3, emb, ln_g, ln_b, blk):
    n = acc2.shape[1]

    def body(a_ref, w_ref, e_ref, g_ref, b_ref, o_ref):
        x = a_ref[0] + a_ref[1]
        h = jnp.maximum(_dotT(x, w_ref[...]), 0.0)
        e = e_ref[...]
        hr = (1.0 - 0.7) * e + 0.7 * h
        m = jnp.mean(hr, axis=-1, keepdims=True)
        v = jnp.mean((hr - m) ** 2, axis=-1, keepdims=True)
        wh = (hr - m) * lax.rsqrt(v + 1e-5) * g_ref[...] + b_ref[...]
        o_ref[...] = wh + e

    return pl.pallas_call(
        body,
        grid=(n // blk,),
        in_specs=[pl.BlockSpec((2, blk, _HID), lambda i: (0, i, 0)),
                  pl.BlockSpec((_HID, _HID), lambda i: (0, 0)),
                  pl.BlockSpec((blk, _HID), lambda i: (i, 0)),
                  pl.BlockSpec((1, _HID), lambda i: (0, 0)),
                  pl.BlockSpec((1, _HID), lambda i: (0, 0))],
        out_specs=pl.BlockSpec((blk, _HID), lambda i: (i, 0)),
        out_shape=jax.ShapeDtypeStruct((n, _HID), jnp.float32),
    )(acc2, w3, emb, ln_g, ln_b)


def _mlp(doc, wm1, bm1, g1, b1, m1, v1, wm2, bm2, g2, b2, m2, v2, wc, bc,
         blk):
    dn = doc.shape[0]
    hh = _HID // 2

    def body(x_ref, wm1r, bm1r, g1r, b1r, m1r, v1r,
             wm2r, bm2r, g2r, b2r, m2r, v2r, wcr, bcr, o_ref):
        x = x_ref[...]
        z = _dotT(x, wm1r[...]) + bm1r[...]
        t = jnp.maximum((z - m1r[...]) * lax.rsqrt(v1r[...] + 1e-5)
                        * g1r[...] + b1r[...], 0.0)
        z2 = _dotT(t, wm2r[...]) + bm2r[...]
        t2 = jnp.maximum((z2 - m2r[...]) * lax.rsqrt(v2r[...] + 1e-5)
                         * g2r[...] + b2r[...], 0.0)
        o_ref[...] = _dotT(t2, wcr[...]) + bcr[...]

    full = lambda shape: pl.BlockSpec(shape, lambda i: tuple(0 for _ in shape))
    return pl.pallas_call(
        body,
        grid=(dn // blk,),
        in_specs=[pl.BlockSpec((blk, _HID), lambda i: (i, 0)),
                  full((_HID, _HID)), full((1, _HID)), full((1, _HID)),
                  full((1, _HID)), full((1, _HID)), full((1, _HID)),
                  full((hh, _HID)), full((1, hh)), full((1, hh)),
                  full((1, hh)), full((1, hh)), full((1, hh)),
                  full((2, hh)), full((1, 2))],
        out_specs=pl.BlockSpec((blk, 2), lambda i: (i, 0)),
        out_shape=jax.ShapeDtypeStruct((dn, 2), jnp.float32),
    )(doc, wm1, bm1, g1, b1, m1, v1, wm2, bm2, g2, b2, m2, v2, wc, bc)


def kernel(a_row, a_col, a_val, x_row, x_col, x_val, emb, W1, W2, W3,
           ln_g, ln_b, Wm1, bm1, bn1_g, bn1_b, bn1_m, bn1_v,
           Wm2, bm2, bn2_g, bn2_b, bn2_m, bn2_v, Wc, bc):
    i32 = jnp.int32
    n, _ = emb.shape
    e = a_row.shape[0]
    nnzx = x_row.shape[0]
    d = 16384
    dh = d // 2

    unit = 2 * _NSUB * _WIN
    e_pad = ((e + unit - 1) // unit) * unit
    pe = e_pad - e
    zi = jnp.arange(pe, dtype=i32) % jnp.int32(n)
    ar2 = jnp.concatenate([a_row.astype(i32), zi]).reshape(e_pad // _CH, _CH)
    ac2 = jnp.concatenate([a_col.astype(i32), zi]).reshape(e_pad // _CH, _CH)
    av2 = jnp.concatenate([a_val, jnp.zeros((pe,), jnp.float32)]
                          ).reshape(e_pad // _CH, _CH)

    x_pad = ((nnzx + _WIN - 1) // _WIN) * _WIN
    px = x_pad - nnzx
    xr = jnp.concatenate([x_row.astype(i32), jnp.full((px,), d - 1, i32)])
    xr2 = xr.reshape(x_pad // _CH, _CH)
    xc2 = jnp.concatenate([x_col.astype(i32), jnp.zeros((px,), i32)]
                          ).reshape(x_pad // _CH, _CH)
    xv2 = jnp.concatenate([x_val, jnp.zeros((px,), jnp.float32)]
                          ).reshape(x_pad // _CH, _CH)

    p = jnp.searchsorted(xr, dh).astype(i32)
    nw = x_pad // _WIN
    bounds = jnp.zeros((2, 16), i32)
    bounds = bounds.at[0, 1].set((p + _WIN - 1) // _WIN)
    bounds = bounds.at[1, 0].set(p // _WIN)
    bounds = bounds.at[1, 1].set(jnp.int32(nw))

    spmm_a = _make_spmm_a(n, e_pad)
    spmm_x = _make_spmm_x(dh, x_pad)

    acc = spmm_a(emb, ar2, ac2, av2)
    h = _mm_relu(acc, W1, 2000)
    acc = spmm_a(h, ar2, ac2, av2)
    h = _mm_relu(acc, W2, 2000)
    acc = spmm_a(h, ar2, ac2, av2)
    s = _post(acc, W3, emb, ln_g.reshape(1, -1), ln_b.reshape(1, -1), 2000)

    dacc = spmm_x(s, xr2, xc2, xv2, bounds)
    doc = dacc.reshape(d, _HID)
    return _mlp(doc, Wm1, bm1.reshape(1, -1), bn1_g.reshape(1, -1),
                bn1_b.reshape(1, -1), bn1_m.reshape(1, -1),
                bn1_v.reshape(1, -1), Wm2, bm2.reshape(1, -1),
                bn2_g.reshape(1, -1), bn2_b.reshape(1, -1),
                bn2_m.reshape(1, -1), bn2_v.reshape(1, -1),
                Wc, bc.reshape(1, -1), 2048)


# speedup vs baseline: 16.9688x; 1.0095x over previous
import jax
import jax.numpy as jnp
from jax import lax
from jax.experimental import pallas as pl
from jax.experimental.pallas import tpu as pltpu
from jax.experimental.pallas import tpu_sc as plsc

_HID = 128
_CH = 128
_KC = 8
_WIN = _CH * _KC
_NSUB = 16


def _mesh():
    return plsc.VectorSubcoreMesh(core_axis_name="c", subcore_axis_name="s",
                                  num_cores=2, num_subcores=_NSUB)


def _zero_fill(zbuf):
    z16 = jnp.zeros((16,), jnp.float32)

    @pl.loop(0, zbuf.shape[0])
    def _zrow(r):
        for q in range(_HID // 16):
            zbuf[r, pl.ds(q * 16, 16)] = z16


def _scale_chunk(buf, valv, k):
    @pl.loop(0, _CH // 16)
    def _sg(g):
        v16 = valv[k, pl.ds(g * 16, 16)]
        for jj in range(16):
            vb = jnp.broadcast_to(v16[jj], (16,))
            j = g * 16 + jj
            for q in range(_HID // 16):
                buf[j, pl.ds(q * 16, 16)] = buf[j, pl.ds(q * 16, 16)] * vb


def _pipeline_window(table, acc, colv, rowv, valv, bufs, gsems, ssems):
    nb = len(bufs)
    ahead = nb - 1
    cps = {}
    for k in range(min(ahead, _KC)):
        cps[k] = pltpu.async_copy(table.at[colv.at[k]], bufs[k % nb],
                                  gsems[k % nb])
    for k in range(_KC):
        b = k % nb
        cps[k].wait()
        nk = k + ahead
        if nk < _KC:
            nbi = nk % nb
            if nk >= nb:
                pltpu.make_async_copy(bufs[nbi], acc.at[rowv.at[nk - nb]],
                                      ssems[nbi]).wait()
            cps[nk] = pltpu.async_copy(table.at[colv.at[nk]],
                                       bufs[nbi], gsems[nbi])
        _scale_chunk(bufs[b], valv, k)
        pltpu.async_copy(bufs[b], acc.at[rowv.at[k]], ssems[b], add=True)
    for k in range(max(_KC - nb, 0), _KC):
        pltpu.make_async_copy(bufs[k % nb], acc.at[rowv.at[k]],
                              ssems[k % nb]).wait()


def _make_spmm_a(n_dst, nnz_pad):
    per_core_rows = nnz_pad // (2 * _CH)
    per_tile_rows = per_core_rows // _NSUB
    nwin = per_tile_rows // _KC
    rpt = 624
    nzc = rpt // 16

    def body(table, rows2, cols2, vals2, out,
             acc, colv, rowv, valv, b0, b1, zbuf,
             g0, g1, s0, s1):
        c = lax.axis_index("c")
        s = lax.axis_index("s")
        bufs = (b0, b1)
        gsems = (g0, g1)
        ssems = (s0, s1)

        _zero_fill(zbuf)

        @pl.loop(0, nzc)
        def _za(t):
            pltpu.sync_copy(zbuf, acc.at[pl.ds(s * rpt + t * 16, 16), :])

        @pl.when(s == _NSUB - 1)
        def _zrem():
            pltpu.sync_copy(zbuf, acc.at[pl.ds(n_dst - 16, 16), :])

        plsc.subcore_barrier()

        @pl.loop(0, nwin)
        def _win(w):
            crow = pl.multiple_of(
                c * per_core_rows + s * per_tile_rows + w * _KC, 8)
            c1 = pltpu.async_copy(cols2.at[pl.ds(crow, _KC)], colv, g0)
            c2 = pltpu.async_copy(rows2.at[pl.ds(crow, _KC)], rowv, g1)
            c3 = pltpu.async_copy(vals2.at[pl.ds(crow, _KC)], valv, s0)
            c1.wait(); c2.wait(); c3.wait()
            _pipeline_window(table, acc, colv, rowv, valv, bufs, gsems, ssems)

        plsc.subcore_barrier()
        base = pl.multiple_of(s * rpt, 8)
        pltpu.sync_copy(acc.at[pl.ds(base, rpt), :],
                        out.at[c].at[pl.ds(base, rpt), :])

        @pl.when(s == _NSUB - 1)
        def _orem():
            pltpu.sync_copy(acc.at[pl.ds(n_dst - 16, 16), :],
                            out.at[c].at[pl.ds(n_dst - 16, 16), :])

    return pl.kernel(
        body,
        out_type=jax.ShapeDtypeStruct((2, n_dst, _HID), jnp.float32),
        mesh=_mesh(),
        scratch_types=[
            pltpu.VMEM_SHARED((n_dst, _HID), jnp.float32),
            pltpu.VMEM((_KC, _CH), jnp.int32),
            pltpu.VMEM((_KC, _CH), jnp.int32),
            pltpu.VMEM((_KC, _CH), jnp.float32),
            pltpu.VMEM((_CH, _HID), jnp.float32),
            pltpu.VMEM((_CH, _HID), jnp.float32),
            pltpu.VMEM((16, _HID), jnp.float32),
            pltpu.SemaphoreType.DMA, pltpu.SemaphoreType.DMA,
            pltpu.SemaphoreType.DMA, pltpu.SemaphoreType.DMA,
        ],
    )


def _make_spmm_x(d_half, nnz_pad):
    nwin_total = nnz_pad // _WIN
    acc_rows = d_half + 2 * _CH
    rpt = acc_rows // _NSUB
    assert rpt % 16 == 0
    rpt_out = d_half // _NSUB

    def body(table, rows2, cols2, vals2, bounds, out,
             acc, colv, rowv, valv, b0, b1, b2, zbuf, wsm,
             g0, g1, g2, s0, s1, s2):
        c = lax.axis_index("c")
        s = lax.axis_index("s")
        bufs = (b0, b1, b2)
        gsems = (g0, g1, g2)
        ssems = (s0, s1, s2)

        _zero_fill(zbuf)

        @pl.loop(0, rpt // 16)
        def _za(t):
            pltpu.sync_copy(zbuf, acc.at[pl.ds(s * rpt + t * 16, 16), :])

        pltpu.sync_copy(bounds.at[c], wsm)
        plsc.subcore_barrier()

        wvec = wsm[...]
        wlo = wvec[0]
        whi = wvec[1]
        w0 = wlo + s
        nsteps = jnp.maximum((whi - w0 + (_NSUB - 1)) // _NSUB, 0)
        roff = c * d_half

        def step(i, carry):
            w = w0 + i * _NSUB
            crow = pl.multiple_of(w * _KC, 8)
            c1 = pltpu.async_copy(cols2.at[pl.ds(crow, _KC)], colv, g0)
            c2 = pltpu.async_copy(rows2.at[pl.ds(crow, _KC)], rowv, g1)
            c3 = pltpu.async_copy(vals2.at[pl.ds(crow, _KC)], valv, s0)
            c1.wait(); c2.wait(); c3.wait()

            @pl.loop(0, _KC)
            def _remap(k):
                @pl.loop(0, _CH // 16)
                def _rg(g):
                    r16 = rowv[k, pl.ds(g * 16, 16)]
                    loc = r16 - roff
                    ok = (loc >= 0) & (loc < d_half)
                    rowv[k, pl.ds(g * 16, 16)] = jnp.where(ok, loc, d_half)

            _pipeline_window(table, acc, colv, rowv, valv, bufs, gsems, ssems)
            return carry

        lax.fori_loop(0, nsteps, step, 0)

        plsc.subcore_barrier()
        base = pl.multiple_of(s * rpt_out, 8)
        pltpu.sync_copy(acc.at[pl.ds(base, rpt_out), :],
                        out.at[c].at[pl.ds(base, rpt_out), :])

    return pl.kernel(
        body,
        out_type=jax.ShapeDtypeStruct((2, d_half, _HID), jnp.float32),
        mesh=_mesh(),
        scratch_types=[
            pltpu.VMEM_SHARED((d_half + 2 * _CH, _HID), jnp.float32),
            pltpu.VMEM((_KC, _CH), jnp.int32),
            pltpu.VMEM((_KC, _CH), jnp.int32),
            pltpu.VMEM((_KC, _CH), jnp.float32),
            pltpu.VMEM((_CH, _HID), jnp.float32),
            pltpu.VMEM((_CH, _HID), jnp.float32),
            pltpu.VMEM((_CH, _HID), jnp.float32),
            pltpu.VMEM((16, _HID), jnp.float32),
            pltpu.VMEM((16,), jnp.int32),
            pltpu.SemaphoreType.DMA, pltpu.SemaphoreType.DMA,
            pltpu.SemaphoreType.DMA, pltpu.SemaphoreType.DMA,
            pltpu.SemaphoreType.DMA, pltpu.SemaphoreType.DMA,
        ],
    )


def _dotT(x, w):
    return lax.dot_general(x, w, (((1,), (1,)), ((), ())),
                           preferred_element_type=jnp.float32)


def _mm_relu(acc2, w, blk):
    n = acc2.shape[1]

    def body(a_ref, w_ref, o_ref):
        x = a_ref[0] + a_ref[1]
        o_ref[...] = jnp.maximum(_dotT(x, w_ref[...]), 0.0)

    return pl.pallas_call(
        body,
        grid=(n // blk,),
        in_specs=[pl.BlockSpec((2, blk, _HID), lambda i: (0, i, 0)),
                  pl.BlockSpec((_HID, _HID), lambda i: (0, 0))],
        out_specs=pl.BlockSpec((blk, _HID), lambda i: (i, 0)),
        out_shape=jax.ShapeDtypeStruct((n, _HID), jnp.float32),
    )(acc2, w)


def _post(acc2, w3, emb, ln_g, ln_b, blk):
    n = acc2.shape[1]

    def body(a_ref, w_ref, e_ref, g_ref, b_ref, o_ref):
        x = a_ref[0] + a_ref[1]
        h = jnp.maximum(_dotT(x, w_ref[...]), 0.0)
        e = e_ref[...]
        hr = (1.0 - 0.7) * e + 0.7 * h
        m = jnp.mean(hr, axis=-1, keepdims=True)
        v = jnp.mean((hr - m) ** 2, axis=-1, keepdims=True)
        wh = (hr - m) * lax.rsqrt(v + 1e-5) * g_ref[...] + b_ref[...]
        o_ref[...] = wh + e

    return pl.pallas_call(
        body,
        grid=(n // blk,),
        in_specs=[pl.BlockSpec((2, blk, _HID), lambda i: (0, i, 0)),
                  pl.BlockSpec((_HID, _HID), lambda i: (0, 0)),
                  pl.BlockSpec((blk, _HID), lambda i: (i, 0)),
                  pl.BlockSpec((1, _HID), lambda i: (0, 0)),
                  pl.BlockSpec((1, _HID), lambda i: (0, 0))],
        out_specs=pl.BlockSpec((blk, _HID), lambda i: (i, 0)),
        out_shape=jax.ShapeDtypeStruct((n, _HID), jnp.float32),
    )(acc2, w3, emb, ln_g, ln_b)


def _mlp(doc, wm1, bm1, g1, b1, m1, v1, wm2, bm2, g2, b2, m2, v2, wc, bc,
         blk):
    dn = doc.shape[0]
    hh = _HID // 2

    def body(x_ref, wm1r, bm1r, g1r, b1r, m1r, v1r,
             wm2r, bm2r, g2r, b2r, m2r, v2r, wcr, bcr, o_ref):
        x = x_ref[...]
        z = _dotT(x, wm1r[...]) + bm1r[...]
        t = jnp.maximum((z - m1r[...]) * lax.rsqrt(v1r[...] + 1e-5)
                        * g1r[...] + b1r[...], 0.0)
        z2 = _dotT(t, wm2r[...]) + bm2r[...]
        t2 = jnp.maximum((z2 - m2r[...]) * lax.rsqrt(v2r[...] + 1e-5)
                         * g2r[...] + b2r[...], 0.0)
        o_ref[...] = _dotT(t2, wcr[...]) + bcr[...]

    full = lambda shape: pl.BlockSpec(shape, lambda i: tuple(0 for _ in shape))
    return pl.pallas_call(
        body,
        grid=(dn // blk,),
        in_specs=[pl.BlockSpec((blk, _HID), lambda i: (i, 0)),
                  full((_HID, _HID)), full((1, _HID)), full((1, _HID)),
                  full((1, _HID)), full((1, _HID)), full((1, _HID)),
                  full((hh, _HID)), full((1, hh)), full((1, hh)),
                  full((1, hh)), full((1, hh)), full((1, hh)),
                  full((2, hh)), full((1, 2))],
        out_specs=pl.BlockSpec((blk, 2), lambda i: (i, 0)),
        out_shape=jax.ShapeDtypeStruct((dn, 2), jnp.float32),
    )(doc, wm1, bm1, g1, b1, m1, v1, wm2, bm2, g2, b2, m2, v2, wc, bc)


def kernel(a_row, a_col, a_val, x_row, x_col, x_val, emb, W1, W2, W3,
           ln_g, ln_b, Wm1, bm1, bn1_g, bn1_b, bn1_m, bn1_v,
           Wm2, bm2, bn2_g, bn2_b, bn2_m, bn2_v, Wc, bc):
    i32 = jnp.int32
    n, _ = emb.shape
    e = a_row.shape[0]
    nnzx = x_row.shape[0]
    d = 16384
    dh = d // 2

    unit = 2 * _NSUB * _WIN
    e_pad = ((e + unit - 1) // unit) * unit
    pe = e_pad - e
    zi = jnp.arange(pe, dtype=i32) % jnp.int32(n)
    ar2 = jnp.concatenate([a_row.astype(i32), zi]).reshape(e_pad // _CH, _CH)
    ac2 = jnp.concatenate([a_col.astype(i32), zi]).reshape(e_pad // _CH, _CH)
    av2 = jnp.concatenate([a_val, jnp.zeros((pe,), jnp.float32)]
                          ).reshape(e_pad // _CH, _CH)

    x_pad = ((nnzx + _WIN - 1) // _WIN) * _WIN
    px = x_pad - nnzx
    xr = jnp.concatenate([x_row.astype(i32), jnp.full((px,), d - 1, i32)])

    def _il(a):
        return a.reshape(-1, 8, 16).swapaxes(1, 2).reshape(x_pad // _CH, _CH)

    xr2 = _il(xr)
    xc2 = _il(jnp.concatenate([x_col.astype(i32), jnp.zeros((px,), i32)]))
    xv2 = _il(jnp.concatenate([x_val, jnp.zeros((px,), jnp.float32)]))

    p = jnp.searchsorted(xr, dh).astype(i32)
    nw = x_pad // _WIN
    bounds = jnp.zeros((2, 16), i32)
    bounds = bounds.at[0, 1].set((p + _WIN - 1) // _WIN)
    bounds = bounds.at[1, 0].set(p // _WIN)
    bounds = bounds.at[1, 1].set(jnp.int32(nw))

    spmm_a = _make_spmm_a(n, e_pad)
    spmm_x = _make_spmm_x(dh, x_pad)

    acc = spmm_a(emb, ar2, ac2, av2)
    h = _mm_relu(acc, W1, 2000)
    acc = spmm_a(h, ar2, ac2, av2)
    h = _mm_relu(acc, W2, 2000)
    acc = spmm_a(h, ar2, ac2, av2)
    s = _post(acc, W3, emb, ln_g.reshape(1, -1), ln_b.reshape(1, -1), 2000)

    dacc = spmm_x(s, xr2, xc2, xv2, bounds)
    doc = dacc.reshape(d, _HID)
    return _mlp(doc, Wm1, bm1.reshape(1, -1), bn1_g.reshape(1, -1),
                bn1_b.reshape(1, -1), bn1_m.reshape(1, -1),
                bn1_v.reshape(1, -1), Wm2, bm2.reshape(1, -1),
                bn2_g.reshape(1, -1), bn2_b.reshape(1, -1),
                bn2_m.reshape(1, -1), bn2_v.reshape(1, -1),
                Wc, bc.reshape(1, -1), 2048)
